# baseline (speedup 1.0000x reference)
import functools

import jax
import jax.numpy as jnp
from jax.experimental import pallas as pl
from jax.experimental.pallas import tpu as pltpu
from jax.experimental.pallas import tpu_sc as plsc

K = 8192
D = 32
TN = 512
TKL = 128
GW = 128
NC = 2
NS = 16
NW = NC * NS
DP = 128


def _argmax_kernel(x_ref, embt_ref, idx_ref):
    x = x_ref[...]
    norm = jnp.sqrt(jnp.sum(x * x, axis=1, keepdims=True))
    xn = x / jnp.maximum(norm, 1e-8)
    dots = jnp.dot(xn, embt_ref[...], preferred_element_type=jnp.float32)
    n_tiles = K // TKL
    best_v = dots[:, :TKL]
    best_t = jnp.zeros((TN, TKL), jnp.int32)
    for t in range(1, n_tiles):
        d = dots[:, t * TKL:(t + 1) * TKL]
        gt = d > best_v
        best_v = jnp.maximum(d, best_v)
        best_t = jnp.where(gt, t * TKL, best_t)
    lane = jax.lax.broadcasted_iota(jnp.int32, (TN, TKL), 1)
    full_idx = best_t + lane
    row_max = jnp.max(best_v, axis=1, keepdims=True)
    cand = jnp.where(best_v == row_max, full_idx, K)
    idx_ref[...] = jnp.min(cand, axis=1)


def _sc_gather_hist(emb_pad, idx, zeros_k, ones_w, n):
    b_per_w = n // NW
    half = b_per_w // 2
    n_chunks = half // GW
    mesh = plsc.VectorSubcoreMesh(core_axis_name="c", subcore_axis_name="s")

    @functools.partial(
        pl.kernel,
        out_type=[
            jax.ShapeDtypeStruct((n, DP), jnp.float32),
            jax.ShapeDtypeStruct((NC, K), jnp.float32),
        ],
        mesh=mesh,
        scratch_types=[
            pltpu.VMEM((b_per_w,), jnp.int32),
            pltpu.VMEM((half, DP), jnp.float32),
            pltpu.VMEM((b_per_w,), jnp.float32),
            pltpu.VMEM_SHARED((K,), jnp.float32),
            pltpu.SemaphoreType.DMA,
            pltpu.SemaphoreType.DMA,
        ],
    )
    def gather_kernel(emb_hbm, idx_hbm, zeros_hbm, ones_hbm, out_hbm,
                      usage_hbm, idx_v, rows_v, ones_v, usage_sh, sem,
                      hsem):
        cid = jax.lax.axis_index("c")
        sid = jax.lax.axis_index("s")
        wid = sid * NC + cid
        base = wid * b_per_w
        pltpu.sync_copy(idx_hbm.at[pl.ds(base, b_per_w)], idx_v)
        pltpu.sync_copy(ones_hbm, ones_v)

        @pl.when(sid == 0)
        def _zero():
            pltpu.sync_copy(zeros_hbm, usage_sh)

        plsc.subcore_barrier()
        hist_copies = [
            pltpu.async_copy(
                ones_v.at[pl.ds(c * GW, GW)],
                usage_sh.at[idx_v.at[pl.ds(c * GW, GW)]],
                hsem,
                add=True,
            )
            for c in range(b_per_w // GW)
        ]
        for h in range(2):
            copies = [
                pltpu.async_copy(
                    emb_hbm.at[idx_v.at[pl.ds(h * half + c * GW, GW)]],
                    rows_v.at[pl.ds(c * GW, GW), :],
                    sem,
                )
                for c in range(n_chunks)
            ]
            for cp in copies:
                cp.wait()
            pltpu.sync_copy(rows_v, out_hbm.at[pl.ds(base + h * half, half)])

        for cp in hist_copies:
            cp.wait()
        plsc.subcore_barrier()

        @pl.when(sid == 0)
        def _write_usage():
            pltpu.sync_copy(usage_sh, usage_hbm.at[cid])

    return gather_kernel(emb_pad, idx, zeros_k, ones_w)


def _finish_kernel(usage2_ref, zqp_ref, zq_ref, stats_ref):
    i = pl.program_id(0)
    zq_ref[...] = zqp_ref[:, :D]

    @pl.when(i == 0)
    def _stats():
        usage = usage2_ref[0, :] + usage2_ref[1, :]
        total = jnp.sum(usage)
        probs = usage / jnp.maximum(total, 1.0)
        safe = jnp.where(probs > 0, probs, 1.0)
        perp = jnp.exp(-jnp.sum(probs * jnp.log(safe)))
        dead = jnp.mean((usage == 0).astype(jnp.float32))
        sel = jax.lax.broadcasted_iota(jnp.int32, (2,), 0) == 0
        stats_ref[...] = jnp.where(sel, perp, dead)


def kernel(z_e, embedding):
    B, L, Dv = z_e.shape
    N = B * L
    n_steps = N // TN
    flat = z_e.reshape(N, Dv)
    embt = embedding.T

    idx = pl.pallas_call(
        _argmax_kernel,
        grid_spec=pl.GridSpec(
            grid=(n_steps,),
            in_specs=[
                pl.BlockSpec((TN, Dv), lambda i: (i, 0)),
                pl.BlockSpec((Dv, K), lambda i: (0, 0)),
            ],
            out_specs=pl.BlockSpec((TN,), lambda i: (i,)),
        ),
        out_shape=jax.ShapeDtypeStruct((N,), jnp.int32),
    )(flat, embt)

    emb_pad = jnp.pad(embedding, ((0, 0), (0, DP - Dv)))
    zeros_k = jnp.zeros((K,), jnp.float32)
    ones_w = jnp.ones((N // NW,), jnp.float32)
    zq_pad, usage2 = _sc_gather_hist(emb_pad, idx, zeros_k, ones_w, N)

    zq_flat, stats = pl.pallas_call(
        _finish_kernel,
        grid_spec=pl.GridSpec(
            grid=(n_steps,),
            in_specs=[
                pl.BlockSpec((NC, K), lambda i: (0, 0)),
                pl.BlockSpec((TN, DP), lambda i: (i, 0)),
            ],
            out_specs=[
                pl.BlockSpec((TN, Dv), lambda i: (i, 0)),
                pl.BlockSpec((2,), lambda i: (0,)),
            ],
        ),
        out_shape=[
            jax.ShapeDtypeStruct((N, Dv), jnp.float32),
            jax.ShapeDtypeStruct((2,), jnp.float32),
        ],
    )(usage2, zq_pad)
    zq = zq_flat.reshape(B, L, Dv)

    return (zq, zq, idx.reshape(B, L), stats)


# speedup vs baseline: 1.1507x; 1.1507x over previous
import functools

import jax
import jax.numpy as jnp
from jax.experimental import pallas as pl
from jax.experimental.pallas import tpu as pltpu
from jax.experimental.pallas import tpu_sc as plsc

K = 8192
D = 32
TN = 512
TKL = 128
GW = 128
NC = 2
NS = 16
NW = NC * NS
DP = 128


def _argmax_kernel(x_ref, embt_ref, idx_ref):
    x = x_ref[...]
    norm = jnp.sqrt(jnp.sum(x * x, axis=1, keepdims=True))
    xn = x / jnp.maximum(norm, 1e-8)
    dots = jnp.dot(xn, embt_ref[...], preferred_element_type=jnp.float32)
    n_tiles = K // TKL
    best_v = dots[:, :TKL]
    best_t = jnp.zeros((TN, TKL), jnp.int32)
    for t in range(1, n_tiles):
        d = dots[:, t * TKL:(t + 1) * TKL]
        gt = d > best_v
        best_v = jnp.maximum(d, best_v)
        best_t = jnp.where(gt, t * TKL, best_t)
    lane = jax.lax.broadcasted_iota(jnp.int32, (TN, TKL), 1)
    full_idx = best_t + lane
    row_max = jnp.max(best_v, axis=1, keepdims=True)
    cand = jnp.where(best_v == row_max, full_idx, K)
    idx_ref[...] = jnp.min(cand, axis=1)


def _sc_gather_hist(emb_pad, idx, zeros_k, ones_w, n):
    b_per_w = n // NW
    half = b_per_w // 2
    n_chunks = half // GW
    mesh = plsc.VectorSubcoreMesh(core_axis_name="c", subcore_axis_name="s")

    @functools.partial(
        pl.kernel,
        out_type=[
            jax.ShapeDtypeStruct((n, DP), jnp.float32),
            jax.ShapeDtypeStruct((NC, K), jnp.float32),
        ],
        mesh=mesh,
        scratch_types=[
            pltpu.VMEM((b_per_w,), jnp.int32),
            pltpu.VMEM((half, DP), jnp.float32),
            pltpu.VMEM((b_per_w,), jnp.float32),
            pltpu.VMEM_SHARED((K,), jnp.float32),
            pltpu.SemaphoreType.DMA,
            pltpu.SemaphoreType.DMA,
        ],
    )
    def gather_kernel(emb_hbm, idx_hbm, zeros_hbm, ones_hbm, out_hbm,
                      usage_hbm, idx_v, rows_v, ones_v, usage_sh, sem,
                      hsem):
        cid = jax.lax.axis_index("c")
        sid = jax.lax.axis_index("s")
        wid = sid * NC + cid
        base = wid * b_per_w
        pltpu.sync_copy(idx_hbm.at[pl.ds(base, b_per_w)], idx_v)
        pltpu.sync_copy(ones_hbm, ones_v)

        @pl.when(sid == 0)
        def _zero():
            pltpu.sync_copy(zeros_hbm, usage_sh)

        plsc.subcore_barrier()
        hist_copies = [
            pltpu.async_copy(
                ones_v.at[pl.ds(c * GW, GW)],
                usage_sh.at[idx_v.at[pl.ds(c * GW, GW)]],
                hsem,
                add=True,
            )
            for c in range(b_per_w // GW)
        ]
        for h in range(2):
            copies = [
                pltpu.async_copy(
                    emb_hbm.at[idx_v.at[pl.ds(h * half + c * GW, GW)]],
                    rows_v.at[pl.ds(c * GW, GW), :],
                    sem,
                )
                for c in range(n_chunks)
            ]
            for cp in copies:
                cp.wait()
            pltpu.sync_copy(rows_v, out_hbm.at[pl.ds(base + h * half, half)])

        for cp in hist_copies:
            cp.wait()
        plsc.subcore_barrier()

        @pl.when(sid == 0)
        def _write_usage():
            pltpu.sync_copy(usage_sh, usage_hbm.at[cid])

    return gather_kernel(emb_pad, idx, zeros_k, ones_w)


def _stats_kernel(usage2_ref, stats_ref):
    usage = usage2_ref[0, :] + usage2_ref[1, :]
    total = jnp.sum(usage)
    probs = usage / jnp.maximum(total, 1.0)
    safe = jnp.where(probs > 0, probs, 1.0)
    perp = jnp.exp(-jnp.sum(probs * jnp.log(safe)))
    dead = jnp.mean((usage == 0).astype(jnp.float32))
    sel = jax.lax.broadcasted_iota(jnp.int32, (2,), 0) == 0
    stats_ref[...] = jnp.where(sel, perp, dead)


def kernel(z_e, embedding):
    B, L, Dv = z_e.shape
    N = B * L
    n_steps = N // TN
    flat = z_e.reshape(N, Dv)
    embt = embedding.T

    idx = pl.pallas_call(
        _argmax_kernel,
        grid_spec=pl.GridSpec(
            grid=(n_steps,),
            in_specs=[
                pl.BlockSpec((TN, Dv), lambda i: (i, 0)),
                pl.BlockSpec((Dv, K), lambda i: (0, 0)),
            ],
            out_specs=pl.BlockSpec((TN,), lambda i: (i,)),
        ),
        out_shape=jax.ShapeDtypeStruct((N,), jnp.int32),
    )(flat, embt)

    emb_pad = jnp.pad(embedding, ((0, 0), (0, DP - Dv)))
    zeros_k = jnp.zeros((K,), jnp.float32)
    ones_w = jnp.ones((N // NW,), jnp.float32)
    zq_pad, usage2 = _sc_gather_hist(emb_pad, idx, zeros_k, ones_w, N)
    zq = zq_pad[:, :Dv].reshape(B, L, Dv)

    stats = pl.pallas_call(
        _stats_kernel,
        out_shape=jax.ShapeDtypeStruct((2,), jnp.float32),
    )(usage2)

    return (zq, zq, idx.reshape(B, L), stats)


# speedup vs baseline: 1.2282x; 1.0673x over previous
import functools

import jax
import jax.numpy as jnp
from jax.experimental import pallas as pl
from jax.experimental.pallas import tpu as pltpu
from jax.experimental.pallas import tpu_sc as plsc

K = 8192
D = 32
TN = 512
TKL = 128
GW = 128
NC = 2
NS = 16
NW = NC * NS
DP = 128


def _argmax_kernel(x_ref, embt_ref, idx_ref):
    x = x_ref[...]
    norm = jnp.sqrt(jnp.sum(x * x, axis=1, keepdims=True))
    xn = x / jnp.maximum(norm, 1e-8)
    n_tiles = K // TKL
    best_v = jnp.dot(xn, embt_ref[:, :TKL],
                     preferred_element_type=jnp.float32)
    best_t = jnp.zeros((TN, TKL), jnp.int32)
    for t in range(1, n_tiles):
        d = jnp.dot(xn, embt_ref[:, t * TKL:(t + 1) * TKL],
                    preferred_element_type=jnp.float32)
        gt = d > best_v
        best_v = jnp.maximum(d, best_v)
        best_t = jnp.where(gt, t * TKL, best_t)
    lane = jax.lax.broadcasted_iota(jnp.int32, (TN, TKL), 1)
    full_idx = best_t + lane
    row_max = jnp.max(best_v, axis=1, keepdims=True)
    cand = jnp.where(best_v == row_max, full_idx, K)
    idx_ref[...] = jnp.min(cand, axis=1)


def _sc_gather_hist(emb_pad, idx, zeros_k, ones_w, n):
    b_per_w = n // NW
    half = b_per_w // 2
    n_chunks = half // GW
    mesh = plsc.VectorSubcoreMesh(core_axis_name="c", subcore_axis_name="s")

    @functools.partial(
        pl.kernel,
        out_type=[
            jax.ShapeDtypeStruct((n, DP), jnp.float32),
            jax.ShapeDtypeStruct((NC, K), jnp.float32),
        ],
        mesh=mesh,
        scratch_types=[
            pltpu.VMEM((b_per_w,), jnp.int32),
            pltpu.VMEM((half, DP), jnp.float32),
            pltpu.VMEM((b_per_w,), jnp.float32),
            pltpu.VMEM_SHARED((K,), jnp.float32),
            pltpu.SemaphoreType.DMA,
            pltpu.SemaphoreType.DMA,
        ],
    )
    def gather_kernel(emb_hbm, idx_hbm, zeros_hbm, ones_hbm, out_hbm,
                      usage_hbm, idx_v, rows_v, ones_v, usage_sh, sem,
                      hsem):
        cid = jax.lax.axis_index("c")
        sid = jax.lax.axis_index("s")
        wid = sid * NC + cid
        base = wid * b_per_w
        pltpu.sync_copy(idx_hbm.at[pl.ds(base, b_per_w)], idx_v)
        pltpu.sync_copy(ones_hbm, ones_v)

        @pl.when(sid == 0)
        def _zero():
            pltpu.sync_copy(zeros_hbm, usage_sh)

        plsc.subcore_barrier()
        hist_copies = [
            pltpu.async_copy(
                ones_v.at[pl.ds(c * GW, GW)],
                usage_sh.at[idx_v.at[pl.ds(c * GW, GW)]],
                hsem,
                add=True,
            )
            for c in range(b_per_w // GW)
        ]
        for h in range(2):
            copies = [
                pltpu.async_copy(
                    emb_hbm.at[idx_v.at[pl.ds(h * half + c * GW, GW)]],
                    rows_v.at[pl.ds(c * GW, GW), :],
                    sem,
                )
                for c in range(n_chunks)
            ]
            for cp in copies:
                cp.wait()
            pltpu.sync_copy(rows_v, out_hbm.at[pl.ds(base + h * half, half)])

        for cp in hist_copies:
            cp.wait()
        plsc.subcore_barrier()

        @pl.when(sid == 0)
        def _write_usage():
            pltpu.sync_copy(usage_sh, usage_hbm.at[cid])

    return gather_kernel(emb_pad, idx, zeros_k, ones_w)


def _stats_kernel(usage2_ref, stats_ref):
    usage = usage2_ref[0, :] + usage2_ref[1, :]
    total = jnp.sum(usage)
    probs = usage / jnp.maximum(total, 1.0)
    safe = jnp.where(probs > 0, probs, 1.0)
    perp = jnp.exp(-jnp.sum(probs * jnp.log(safe)))
    dead = jnp.mean((usage == 0).astype(jnp.float32))
    sel = jax.lax.broadcasted_iota(jnp.int32, (2,), 0) == 0
    stats_ref[...] = jnp.where(sel, perp, dead)


def kernel(z_e, embedding):
    B, L, Dv = z_e.shape
    N = B * L
    n_steps = N // TN
    flat = z_e.reshape(N, Dv)
    embt = embedding.T

    idx = pl.pallas_call(
        _argmax_kernel,
        grid_spec=pl.GridSpec(
            grid=(n_steps,),
            in_specs=[
                pl.BlockSpec((TN, Dv), lambda i: (i, 0)),
                pl.BlockSpec((Dv, K), lambda i: (0, 0)),
            ],
            out_specs=pl.BlockSpec((TN,), lambda i: (i,)),
        ),
        out_shape=jax.ShapeDtypeStruct((N,), jnp.int32),
    )(flat, embt)

    emb_pad = jnp.pad(embedding, ((0, 0), (0, DP - Dv)))
    zeros_k = jnp.zeros((K,), jnp.float32)
    ones_w = jnp.ones((N // NW,), jnp.float32)
    zq_pad, usage2 = _sc_gather_hist(emb_pad, idx, zeros_k, ones_w, N)
    zq = zq_pad[:, :Dv].reshape(B, L, Dv)

    stats = pl.pallas_call(
        _stats_kernel,
        out_shape=jax.ShapeDtypeStruct((2,), jnp.float32),
    )(usage2)

    return (zq, zq, idx.reshape(B, L), stats)
